# Initial kernel scaffold; baseline (speedup 1.0000x reference)
import jax
import jax.numpy as jnp
from jax.experimental import pallas as pl


def kernel(x, W):
    raise NotImplementedError("write your pallas kernel here")


# speedup vs baseline: 4.2446x; 4.2446x over previous
import functools

import jax
import jax.numpy as jnp
from jax.experimental import pallas as pl
from jax.experimental.pallas import tpu as pltpu

N_EMBD = 2048
N_EXP = 64
TOP_K = 2
N_GROUPS = 8
EXP_PER_GROUP = N_EXP // N_GROUPS
N_LIMITED_GROUPS = 4

_BLK = 512


def _router_body(x_ref, w_ref, wts_ref, idx_ref, cnt_ref):
    b = x_ref.shape[0]
    z = jnp.dot(x_ref[...], w_ref[...], preferred_element_type=jnp.float32)
    e = jnp.exp(-jnp.abs(z))
    s = jnp.where(z >= 0, 1.0 / (1.0 + e), e / (1.0 + e))

    lane = jax.lax.broadcasted_iota(jnp.int32, (b, N_EXP), 1)
    grp_of_lane = lane // EXP_PER_GROUP
    neg_inf = jnp.float32(-jnp.inf)

    gmax = []
    for g in range(N_GROUPS):
        in_g = grp_of_lane == g
        gmax.append(
            jnp.max(jnp.where(in_g, s, neg_inf), axis=1, keepdims=True))

    sel = []
    for g in range(N_GROUPS):
        rank = jnp.zeros((b, 1), jnp.int32)
        for j in range(N_GROUPS):
            if j == g:
                continue
            if j < g:
                beats = gmax[j] >= gmax[g]
            else:
                beats = gmax[j] > gmax[g]
            rank = rank + beats.astype(jnp.int32)
        sel.append(rank < N_LIMITED_GROUPS)

    keep = jnp.zeros((b, N_EXP), jnp.bool_)
    for g in range(N_GROUPS):
        keep = jnp.logical_or(keep, jnp.logical_and(grp_of_lane == g, sel[g]))
    sm = jnp.where(keep, s, neg_inf)

    v1 = jnp.max(sm, axis=1, keepdims=True)
    i1 = jnp.min(jnp.where(sm == v1, lane, N_EXP), axis=1, keepdims=True)
    sm2 = jnp.where(lane == i1, neg_inf, sm)
    v2 = jnp.max(sm2, axis=1, keepdims=True)
    i2 = jnp.min(jnp.where(sm2 == v2, lane, N_EXP), axis=1, keepdims=True)

    den = v1 + v2
    wts_ref[...] = jnp.concatenate([v1 / den, v2 / den], axis=1)
    idx_ref[...] = jnp.concatenate([i1, i2], axis=1)

    hits = (lane == i1).astype(jnp.int32) + (lane == i2).astype(jnp.int32)
    part = jnp.sum(hits, axis=0, keepdims=True)

    @pl.when(pl.program_id(0) == 0)
    def _init():
        cnt_ref[...] = jnp.zeros_like(cnt_ref)

    cnt_ref[...] += part


@jax.jit
def kernel(x, W):
    n_tok = x.shape[0]
    grid = n_tok // _BLK
    wts, idx, cnt = pl.pallas_call(
        _router_body,
        grid=(grid,),
        in_specs=[
            pl.BlockSpec((_BLK, N_EMBD), lambda i: (i, 0)),
            pl.BlockSpec((N_EMBD, N_EXP), lambda i: (0, 0)),
        ],
        out_specs=[
            pl.BlockSpec((_BLK, TOP_K), lambda i: (i, 0)),
            pl.BlockSpec((_BLK, TOP_K), lambda i: (i, 0)),
            pl.BlockSpec((1, N_EXP), lambda i: (0, 0)),
        ],
        out_shape=[
            jax.ShapeDtypeStruct((n_tok, TOP_K), jnp.float32),
            jax.ShapeDtypeStruct((n_tok, TOP_K), jnp.int32),
            jax.ShapeDtypeStruct((1, N_EXP), jnp.int32),
        ],
        compiler_params=pltpu.CompilerParams(
            dimension_semantics=("arbitrary",),
        ),
    )(x, W)
    return wts, idx, cnt.reshape(N_EXP)


# speedup vs baseline: 4.4339x; 1.0446x over previous
import functools

import jax
import jax.numpy as jnp
from jax.experimental import pallas as pl
from jax.experimental.pallas import tpu as pltpu

N_EMBD = 2048
N_EXP = 64
TOP_K = 2
N_GROUPS = 8
EXP_PER_GROUP = N_EXP // N_GROUPS
N_LIMITED_GROUPS = 4

_BLK = 1024


def _router_body(x_ref, w_ref, wts_ref, idx_ref, cnt_ref):
    b = x_ref.shape[0]
    z = jnp.dot(x_ref[...], w_ref[...], preferred_element_type=jnp.float32)
    e = jnp.exp(-jnp.abs(z))
    s = jnp.where(z >= 0, 1.0 / (1.0 + e), e / (1.0 + e))

    lane = jax.lax.broadcasted_iota(jnp.int32, (b, N_EXP), 1)
    grp_of_lane = lane // EXP_PER_GROUP
    neg_inf = jnp.float32(-jnp.inf)

    gmax = []
    for g in range(N_GROUPS):
        in_g = grp_of_lane == g
        gmax.append(
            jnp.max(jnp.where(in_g, s, neg_inf), axis=1, keepdims=True))

    sel = []
    for g in range(N_GROUPS):
        rank = jnp.zeros((b, 1), jnp.int32)
        for j in range(N_GROUPS):
            if j == g:
                continue
            if j < g:
                beats = gmax[j] >= gmax[g]
            else:
                beats = gmax[j] > gmax[g]
            rank = rank + beats.astype(jnp.int32)
        sel.append(rank < N_LIMITED_GROUPS)

    keep = jnp.zeros((b, N_EXP), jnp.bool_)
    for g in range(N_GROUPS):
        keep = jnp.logical_or(keep, jnp.logical_and(grp_of_lane == g, sel[g]))
    sm = jnp.where(keep, s, neg_inf)

    v1 = jnp.max(sm, axis=1, keepdims=True)
    i1 = jnp.min(jnp.where(sm == v1, lane, N_EXP), axis=1, keepdims=True)
    sm2 = jnp.where(lane == i1, neg_inf, sm)
    v2 = jnp.max(sm2, axis=1, keepdims=True)
    i2 = jnp.min(jnp.where(sm2 == v2, lane, N_EXP), axis=1, keepdims=True)

    den = v1 + v2
    wts_ref[...] = jnp.concatenate([v1 / den, v2 / den], axis=1)
    idx_ref[...] = jnp.concatenate([i1, i2], axis=1)

    hits = (lane == i1).astype(jnp.int32) + (lane == i2).astype(jnp.int32)
    part = jnp.sum(hits, axis=0, keepdims=True)

    @pl.when(pl.program_id(0) == 0)
    def _init():
        cnt_ref[...] = jnp.zeros_like(cnt_ref)

    cnt_ref[...] += part


@jax.jit
def kernel(x, W):
    n_tok = x.shape[0]
    grid = n_tok // _BLK
    wts, idx, cnt = pl.pallas_call(
        _router_body,
        grid=(grid,),
        in_specs=[
            pl.BlockSpec((_BLK, N_EMBD), lambda i: (i, 0)),
            pl.BlockSpec((N_EMBD, N_EXP), lambda i: (0, 0)),
        ],
        out_specs=[
            pl.BlockSpec((_BLK, TOP_K), lambda i: (i, 0)),
            pl.BlockSpec((_BLK, TOP_K), lambda i: (i, 0)),
            pl.BlockSpec((1, N_EXP), lambda i: (0, 0)),
        ],
        out_shape=[
            jax.ShapeDtypeStruct((n_tok, TOP_K), jnp.float32),
            jax.ShapeDtypeStruct((n_tok, TOP_K), jnp.int32),
            jax.ShapeDtypeStruct((1, N_EXP), jnp.int32),
        ],
        compiler_params=pltpu.CompilerParams(
            dimension_semantics=("arbitrary",),
        ),
    )(x, W)
    return wts, idx, cnt.reshape(N_EXP)
